# Initial kernel scaffold; baseline (speedup 1.0000x reference)
import jax
import jax.numpy as jnp
from jax.experimental import pallas as pl


def kernel(x, edge_index, W, b, ln_w, ln_b, lin_W, lin_b):
    raise NotImplementedError("write your pallas kernel here")


# speedup vs baseline: 13.1266x; 13.1266x over previous
import functools

import jax
import jax.numpy as jnp
from jax import lax
from jax.experimental import pallas as pl
from jax.experimental.pallas import tpu as pltpu
from jax.experimental.pallas import tpu_sc as plsc

N_NODES = 10000
N_EDGES = 320000
D = 128
EPS = 1e-5

NC = 2
NS = 16
NW = NC * NS

CH = 128
KCH = 80
EPT = KCH * CH
E_PAD = NW * EPT
NPAD = 10112
RPT = NPAD // NS
DW = 16

ROWBLK = 1000
NBLK = N_NODES // ROWBLK

_mesh = plsc.VectorSubcoreMesh(core_axis_name="c", subcore_axis_name="s",
                               num_cores=NC, num_subcores=NS)


@functools.partial(
    pl.kernel,
    out_type=jax.ShapeDtypeStruct((NC, NPAD, DW), jnp.float32),
    mesh=_mesh,
    scratch_types=[
        pltpu.VMEM((KCH, CH), jnp.int32),
        pltpu.VMEM((CH, DW), jnp.float32),
        pltpu.VMEM_SHARED((NPAD, DW), jnp.float32),
    ],
)
def _deg_kernel(dst_hbm, z1_hbm, degp_hbm, dstv, onesv, deg_sh):
    c = lax.axis_index("c")
    s = lax.axis_index("s")
    wid = c * NS + s
    pltpu.sync_copy(dst_hbm.at[wid, 0], dstv.at[pl.ds(0, KCH // 2)])
    pltpu.sync_copy(dst_hbm.at[wid, 1], dstv.at[pl.ds(KCH // 2, KCH // 2)])

    def _fill(r, carry):
        onesv[r, :] = jnp.ones((DW,), jnp.float32)
        return carry

    lax.fori_loop(0, CH, _fill, 0)

    pltpu.sync_copy(z1_hbm.at[pl.ds(s * RPT, RPT)],
                    deg_sh.at[pl.ds(s * RPT, RPT)])
    plsc.subcore_barrier()

    def _scat(j, carry):
        pltpu.sync_copy(onesv, deg_sh.at[dstv.at[j]], add=True)
        return carry

    lax.fori_loop(0, KCH, _scat, 0)

    plsc.subcore_barrier()
    pltpu.sync_copy(deg_sh.at[pl.ds(s * RPT, RPT)],
                    degp_hbm.at[c, pl.ds(s * RPT, RPT)])


def _scale_body(x_ref, degp_ref, g_ref):
    deg = degp_ref[0, :, 0:1] + degp_ref[1, :, 0:1] + 1.0
    dinv = lax.rsqrt(deg)
    g_ref[...] = x_ref[...] * dinv


def _scale(x, degp):
    return pl.pallas_call(
        _scale_body,
        grid=(NBLK,),
        in_specs=[
            pl.BlockSpec((ROWBLK, D), lambda i: (i, 0)),
            pl.BlockSpec((NC, ROWBLK, DW), lambda i: (0, i, 0)),
        ],
        out_specs=pl.BlockSpec((ROWBLK, D), lambda i: (i, 0)),
        out_shape=jax.ShapeDtypeStruct((N_NODES, D), jnp.float32),
    )(x, degp)


@functools.partial(
    pl.kernel,
    out_type=jax.ShapeDtypeStruct((NC, NPAD, D), jnp.float32),
    mesh=_mesh,
    scratch_types=[
        pltpu.VMEM((KCH // 2, CH), jnp.int32),
        pltpu.VMEM((KCH // 2, CH), jnp.int32),
        pltpu.VMEM((CH, D), jnp.float32),
        pltpu.VMEM((CH, D), jnp.float32),
        pltpu.VMEM_SHARED((NPAD, D), jnp.float32),
        pltpu.SemaphoreType.DMA,
        pltpu.SemaphoreType.DMA,
    ],
)
def _gs_kernel(g_hbm, src_hbm, dst_hbm, z2_hbm, outp_hbm,
               srcv, dstv, rows0, rows1, acc_sh, sem0, sem1):
    c = lax.axis_index("c")
    s = lax.axis_index("s")
    wid = c * NS + s
    kh = KCH // 2
    pltpu.sync_copy(z2_hbm.at[pl.ds(s * RPT, RPT)],
                    acc_sh.at[pl.ds(s * RPT, RPT)])
    plsc.subcore_barrier()

    for h in range(2):
        pltpu.sync_copy(src_hbm.at[wid, h], srcv)
        pltpu.sync_copy(dst_hbm.at[wid, h], dstv)

        pltpu.async_copy(g_hbm.at[srcv.at[0]], rows0, sem0)
        pltpu.async_copy(g_hbm.at[srcv.at[1]], rows1, sem1)

        def _step(k, carry):
            j0 = 2 * k
            j1 = j0 + 1
            pltpu.make_async_copy(g_hbm.at[srcv.at[j0]], rows0, sem0).wait()
            pltpu.sync_copy(rows0, acc_sh.at[dstv.at[j0]], add=True)

            @pl.when(j0 + 2 < kh)
            def _():
                pltpu.async_copy(g_hbm.at[srcv.at[j0 + 2]], rows0, sem0)

            pltpu.make_async_copy(g_hbm.at[srcv.at[j1]], rows1, sem1).wait()
            pltpu.sync_copy(rows1, acc_sh.at[dstv.at[j1]], add=True)

            @pl.when(j1 + 2 < kh)
            def _():
                pltpu.async_copy(g_hbm.at[srcv.at[j1 + 2]], rows1, sem1)

            return carry

        lax.fori_loop(0, kh // 2, _step, 0)

    plsc.subcore_barrier()
    pltpu.sync_copy(acc_sh.at[pl.ds(s * RPT, RPT)],
                    outp_hbm.at[c, pl.ds(s * RPT, RPT)])


def _final_body(p_ref, g_ref, degp_ref, W_ref, b_ref, lnw_ref, lnb_ref,
                linW_ref, linb_ref, out_ref):
    gv = g_ref[...]
    acc = p_ref[0] + p_ref[1] + gv
    deg = degp_ref[0, :, 0:1] + degp_ref[1, :, 0:1] + 1.0
    pre = acc * lax.rsqrt(deg)
    y = jnp.dot(pre, W_ref[...], precision=lax.Precision.HIGHEST,
                preferred_element_type=jnp.float32) + b_ref[...]
    y = jnp.maximum(y, 0.0)
    mu = jnp.mean(y, axis=1, keepdims=True)
    var = jnp.mean((y - mu) * (y - mu), axis=1, keepdims=True)
    yn = (y - mu) * lax.rsqrt(var + EPS) * lnw_ref[...] + lnb_ref[...]
    out_ref[...] = jnp.dot(yn, linW_ref[...], precision=lax.Precision.HIGHEST,
                           preferred_element_type=jnp.float32) + linb_ref[...]


def _final(p, g, degp, W, b, ln_w, ln_b, lin_W, lin_b):
    row = lambda i: (i, 0)
    full2 = pl.BlockSpec((1, D), lambda i: (0, 0))
    return pl.pallas_call(
        _final_body,
        grid=(NBLK,),
        in_specs=[
            pl.BlockSpec((NC, ROWBLK, D), lambda i: (0, i, 0)),
            pl.BlockSpec((ROWBLK, D), row),
            pl.BlockSpec((NC, ROWBLK, DW), lambda i: (0, i, 0)),
            pl.BlockSpec((D, D), lambda i: (0, 0)),
            full2, full2, full2,
            pl.BlockSpec((D, D), lambda i: (0, 0)),
            full2,
        ],
        out_specs=pl.BlockSpec((ROWBLK, D), row),
        out_shape=jax.ShapeDtypeStruct((N_NODES, D), jnp.float32),
    )(p, g, degp, W, b, ln_w, ln_b, lin_W, lin_b)


def kernel(x, edge_index, W, b, ln_w, ln_b, lin_W, lin_b):
    ei = edge_index.astype(jnp.int32)
    pad = E_PAD - N_EDGES
    src_p = jnp.concatenate(
        [ei[0], jnp.zeros((pad,), jnp.int32)]).reshape(NW, 2, KCH // 2, CH)
    dst_p = jnp.concatenate(
        [ei[1], jnp.full((pad,), N_NODES, jnp.int32)]).reshape(NW, 2, KCH // 2, CH)
    z1 = jnp.zeros((NPAD, DW), jnp.float32)
    z2 = jnp.zeros((NPAD, D), jnp.float32)

    degp = _deg_kernel(dst_p, z1)
    g = _scale(x, degp)
    p = _gs_kernel(g, src_p, dst_p, z2)
    b2 = b.reshape(1, D)
    return _final(p, g, degp, W, b2, ln_w.reshape(1, D), ln_b.reshape(1, D),
                  lin_W, lin_b.reshape(1, D))


# speedup vs baseline: 34.7414x; 2.6467x over previous
import functools

import jax
import jax.numpy as jnp
from jax import lax
from jax.experimental import pallas as pl
from jax.experimental.pallas import tpu as pltpu
from jax.experimental.pallas import tpu_sc as plsc

N_NODES = 10000
N_EDGES = 320000
D = 128
EPS = 1e-5

NC = 2
NS = 16
NW = NC * NS

CH = 128
KCH = 80
EPT = KCH * CH
E_PAD = NW * EPT
NPAD = 10112
RPT = NPAD // NS
DW = 16

ROWBLK = 1000
NBLK = N_NODES // ROWBLK

_mesh = plsc.VectorSubcoreMesh(core_axis_name="c", subcore_axis_name="s",
                               num_cores=NC, num_subcores=NS)


@functools.partial(
    pl.kernel,
    out_type=jax.ShapeDtypeStruct((NC, NPAD, DW), jnp.float32),
    mesh=_mesh,
    scratch_types=[
        pltpu.VMEM((KCH, CH), jnp.int32),
        pltpu.VMEM((CH, DW), jnp.float32),
        pltpu.VMEM_SHARED((NPAD, DW), jnp.float32),
    ],
)
def _deg_kernel(dst_hbm, z1_hbm, degp_hbm, dstv, onesv, deg_sh):
    c = lax.axis_index("c")
    s = lax.axis_index("s")
    wid = c * NS + s
    pltpu.sync_copy(dst_hbm.at[wid, 0], dstv.at[pl.ds(0, KCH // 2)])
    pltpu.sync_copy(dst_hbm.at[wid, 1], dstv.at[pl.ds(KCH // 2, KCH // 2)])

    def _fill(r, carry):
        onesv[r, :] = jnp.ones((DW,), jnp.float32)
        return carry

    lax.fori_loop(0, CH, _fill, 0)

    pltpu.sync_copy(z1_hbm.at[pl.ds(s * RPT, RPT)],
                    deg_sh.at[pl.ds(s * RPT, RPT)])
    plsc.subcore_barrier()

    def _scat(j, carry):
        pltpu.sync_copy(onesv, deg_sh.at[dstv.at[j]], add=True)
        return carry

    lax.fori_loop(0, KCH, _scat, 0)

    plsc.subcore_barrier()
    pltpu.sync_copy(deg_sh.at[pl.ds(s * RPT, RPT)],
                    degp_hbm.at[c, pl.ds(s * RPT, RPT)])


def _scale_body(x_ref, degp_ref, g_ref):
    deg = degp_ref[0, :, 0:1] + degp_ref[1, :, 0:1] + 1.0
    dinv = lax.rsqrt(deg)
    g_ref[...] = x_ref[...] * dinv


def _scale(x, degp):
    return pl.pallas_call(
        _scale_body,
        grid=(NBLK,),
        in_specs=[
            pl.BlockSpec((ROWBLK, D), lambda i: (i, 0)),
            pl.BlockSpec((NC, ROWBLK, DW), lambda i: (0, i, 0)),
        ],
        out_specs=pl.BlockSpec((ROWBLK, D), lambda i: (i, 0)),
        out_shape=jax.ShapeDtypeStruct((N_NODES, D), jnp.float32),
    )(x, degp)


@functools.partial(
    pl.kernel,
    out_type=jax.ShapeDtypeStruct((NC, NPAD, D), jnp.float32),
    mesh=_mesh,
    scratch_types=[
        pltpu.VMEM((KCH // 2, CH), jnp.int32),
        pltpu.VMEM((KCH // 2, CH), jnp.int32),
        pltpu.VMEM((CH, D), jnp.float32),
        pltpu.VMEM((CH, D), jnp.float32),
        pltpu.VMEM_SHARED((NPAD, D), jnp.float32),
        pltpu.SemaphoreType.DMA,
        pltpu.SemaphoreType.DMA,
    ],
)
def _gs_kernel(g_hbm, src_hbm, dst_hbm, z2_hbm, outp_hbm,
               srcv, dstv, rows0, rows1, acc_sh, sem0, sem1):
    c = lax.axis_index("c")
    s = lax.axis_index("s")
    wid = c * NS + s
    kh = KCH // 2
    pltpu.sync_copy(z2_hbm.at[pl.ds(s * RPT, RPT)],
                    acc_sh.at[pl.ds(s * RPT, RPT)])
    plsc.subcore_barrier()

    for h in range(2):
        pltpu.sync_copy(src_hbm.at[wid, h], srcv)
        pltpu.sync_copy(dst_hbm.at[wid, h], dstv)

        pltpu.async_copy(g_hbm.at[srcv.at[0]], rows0, sem0)
        pltpu.async_copy(g_hbm.at[srcv.at[1]], rows1, sem1)

        def _step(k, carry):
            j0 = 2 * k
            j1 = j0 + 1
            pltpu.make_async_copy(g_hbm.at[srcv.at[j0]], rows0, sem0).wait()
            pltpu.sync_copy(rows0, acc_sh.at[dstv.at[j0]], add=True)

            @pl.when(j0 + 2 < kh)
            def _():
                pltpu.async_copy(g_hbm.at[srcv.at[j0 + 2]], rows0, sem0)

            pltpu.make_async_copy(g_hbm.at[srcv.at[j1]], rows1, sem1).wait()
            pltpu.sync_copy(rows1, acc_sh.at[dstv.at[j1]], add=True)

            @pl.when(j1 + 2 < kh)
            def _():
                pltpu.async_copy(g_hbm.at[srcv.at[j1 + 2]], rows1, sem1)

            return carry

        lax.fori_loop(0, kh // 2, _step, 0)

    plsc.subcore_barrier()
    pltpu.sync_copy(acc_sh.at[pl.ds(s * RPT, RPT)],
                    outp_hbm.at[c, pl.ds(s * RPT, RPT)])


def _final_body(p_ref, g_ref, degp_ref, W_ref, b_ref, lnw_ref, lnb_ref,
                linW_ref, linb_ref, out_ref):
    gv = g_ref[...]
    acc = p_ref[0] + p_ref[1] + gv
    deg = degp_ref[0, :, 0:1] + degp_ref[1, :, 0:1] + 1.0
    pre = acc * lax.rsqrt(deg)
    y = jnp.dot(pre, W_ref[...], precision=lax.Precision.HIGHEST,
                preferred_element_type=jnp.float32) + b_ref[...]
    y = jnp.maximum(y, 0.0)
    mu = jnp.mean(y, axis=1, keepdims=True)
    var = jnp.mean((y - mu) * (y - mu), axis=1, keepdims=True)
    yn = (y - mu) * lax.rsqrt(var + EPS) * lnw_ref[...] + lnb_ref[...]
    out_ref[...] = jnp.dot(yn, linW_ref[...], precision=lax.Precision.HIGHEST,
                           preferred_element_type=jnp.float32) + linb_ref[...]


def _final(p, g, degp, W, b, ln_w, ln_b, lin_W, lin_b):
    row = lambda i: (i, 0)
    full2 = pl.BlockSpec((1, D), lambda i: (0, 0))
    return pl.pallas_call(
        _final_body,
        grid=(NBLK,),
        in_specs=[
            pl.BlockSpec((NC, ROWBLK, D), lambda i: (0, i, 0)),
            pl.BlockSpec((ROWBLK, D), row),
            pl.BlockSpec((NC, ROWBLK, DW), lambda i: (0, i, 0)),
            pl.BlockSpec((D, D), lambda i: (0, 0)),
            full2, full2, full2,
            pl.BlockSpec((D, D), lambda i: (0, 0)),
            full2,
        ],
        out_specs=pl.BlockSpec((ROWBLK, D), row),
        out_shape=jax.ShapeDtypeStruct((N_NODES, D), jnp.float32),
    )(p, g, degp, W, b, ln_w, ln_b, lin_W, lin_b)


def kernel(x, edge_index, W, b, ln_w, ln_b, lin_W, lin_b):
    ei = edge_index.astype(jnp.int32)
    ppt = EPT - N_EDGES // NW
    pad_src = jnp.broadcast_to(jnp.arange(ppt, dtype=jnp.int32) % N_NODES,
                               (NW, ppt))
    pad_dst = jnp.broadcast_to(
        N_NODES + jnp.arange(ppt, dtype=jnp.int32) % (NPAD - N_NODES),
        (NW, ppt))
    src_p = jnp.concatenate(
        [ei[0].reshape(NW, -1), pad_src], axis=1).reshape(NW, 2, KCH // 2, CH)
    dst_p = jnp.concatenate(
        [ei[1].reshape(NW, -1), pad_dst], axis=1).reshape(NW, 2, KCH // 2, CH)
    z1 = jnp.zeros((NPAD, DW), jnp.float32)
    z2 = jnp.zeros((NPAD, D), jnp.float32)

    degp = _deg_kernel(dst_p, z1)
    g = _scale(x, degp)
    p = _gs_kernel(g, src_p, dst_p, z2)
    b2 = b.reshape(1, D)
    return _final(p, g, degp, W, b2, ln_w.reshape(1, D), ln_b.reshape(1, D),
                  lin_W, lin_b.reshape(1, D))
